# baseline (speedup 1.0000x reference)
import functools

import jax
import jax.numpy as jnp
from jax import lax
from jax.experimental import pallas as pl
from jax.experimental.pallas import tpu as pltpu
from jax.experimental.pallas import tpu_sc as plsc

NC, NS, LANES = 2, 16, 16
NW = NC * NS

B, S, D = 4, 2048, 1024
N_ROWS = B * S
S_PER_W = S // NW
S_CHUNK = 32


def _body(x_hbm, table_hbm, pos_hbm, out_hbm, idx_v, pos_v, rows_v, gsem):
    wid = lax.axis_index("s") * NC + lax.axis_index("c")
    s0 = wid * S_PER_W
    for b in range(B):
        pltpu.sync_copy(
            x_hbm.at[pl.ds(b * S + s0, S_PER_W)],
            idx_v.at[pl.ds(b * S_PER_W, S_PER_W)],
        )
    for sub in range(S_PER_W // S_CHUNK):
        pltpu.sync_copy(pos_hbm.at[pl.ds(s0 + sub * S_CHUNK, S_CHUNK)], pos_v)
        for b in range(B):
            idx_slice = idx_v.at[pl.ds(b * S_PER_W + sub * S_CHUNK, S_CHUNK)]
            pltpu.async_copy(table_hbm.at[idx_slice], rows_v, gsem).wait()

            def add_row(r, carry):
                for j in range(D // LANES):
                    sl = pl.ds(j * LANES, LANES)
                    rows_v[r, sl] = rows_v[r, sl] + pos_v[r, sl]
                return carry

            lax.fori_loop(0, S_CHUNK, add_row, 0)
            row0 = b * S + s0 + sub * S_CHUNK
            pltpu.sync_copy(rows_v, out_hbm.at[pl.ds(row0, S_CHUNK)])


_sc_lookup = pl.kernel(
    _body,
    out_type=jax.ShapeDtypeStruct((N_ROWS, D), jnp.float32),
    mesh=plsc.VectorSubcoreMesh(core_axis_name="c", subcore_axis_name="s"),
    scratch_types=[
        pltpu.VMEM((B * S_PER_W,), jnp.int32),
        pltpu.VMEM((S_CHUNK, D), jnp.float32),
        pltpu.VMEM((S_CHUNK, D), jnp.float32),
        pltpu.SemaphoreType.DMA,
    ],
)


@jax.jit
def kernel(x, token_table, pos_emb):
    h = _sc_lookup(x.reshape(N_ROWS), token_table, pos_emb)
    return h.reshape(B, S, D)


# speedup vs baseline: 1.1388x; 1.1388x over previous
import functools

import jax
import jax.numpy as jnp
from jax import lax
from jax.experimental import pallas as pl
from jax.experimental.pallas import tpu as pltpu
from jax.experimental.pallas import tpu_sc as plsc

NC, NS, LANES = 2, 16, 16
NW = NC * NS

B, S, D = 4, 2048, 1024
N_ROWS = B * S
S_PER_W = S // NW
S_CHUNK = 16
N_SUB = S_PER_W // S_CHUNK
NSTEP = N_SUB * B


def _body(x_hbm, table_hbm, pos_hbm, out_hbm,
          idx_v, pos_v, rows_v, gsem0, gsem1, wsem0, wsem1, psem):
    gsems = (gsem0, gsem1)
    wsems = (wsem0, wsem1)
    wid = lax.axis_index("s") * NC + lax.axis_index("c")
    s0 = wid * S_PER_W

    pos_cp = pltpu.async_copy(pos_hbm.at[pl.ds(s0, S_PER_W)], pos_v, psem)
    for b in range(B):
        pltpu.sync_copy(
            x_hbm.at[pl.ds(b * S + s0, S_PER_W)],
            idx_v.at[pl.ds(b * S_PER_W, S_PER_W)],
        )

    def start_gather(step, buf):
        sub, b = divmod(step, B)
        idx_slice = idx_v.at[pl.ds(b * S_PER_W + sub * S_CHUNK, S_CHUNK)]
        return pltpu.async_copy(table_hbm.at[idx_slice], rows_v.at[buf],
                                gsems[buf])

    def start_write(step, buf):
        sub, b = divmod(step, B)
        row0 = b * S + s0 + sub * S_CHUNK
        return pltpu.async_copy(rows_v.at[buf], out_hbm.at[pl.ds(row0, S_CHUNK)],
                                wsems[buf])

    def add_pos(step, buf):
        sub, _ = divmod(step, B)

        def add_row(r, carry):
            for j in range(D // LANES):
                sl = pl.ds(j * LANES, LANES)
                plsc.addupdate(rows_v.at[buf, r, sl], pos_v[sub * S_CHUNK + r, sl])
            return carry

        lax.fori_loop(0, S_CHUNK, add_row, 0)

    g_pending = [None, None]
    w_pending = [None, None]
    g_pending[0] = start_gather(0, 0)
    for step in range(NSTEP):
        buf = step % 2
        nxt = 1 - buf
        if step + 1 < NSTEP:
            if w_pending[nxt] is not None:
                w_pending[nxt].wait()
            g_pending[nxt] = start_gather(step + 1, nxt)
        if step == 0:
            pos_cp.wait()
        g_pending[buf].wait()
        add_pos(step, buf)
        w_pending[buf] = start_write(step, buf)
    for buf in range(2):
        if w_pending[buf] is not None:
            w_pending[buf].wait()


_sc_lookup = pl.kernel(
    _body,
    out_type=jax.ShapeDtypeStruct((N_ROWS, D), jnp.float32),
    mesh=plsc.VectorSubcoreMesh(core_axis_name="c", subcore_axis_name="s"),
    scratch_types=[
        pltpu.VMEM((B * S_PER_W,), jnp.int32),
        pltpu.VMEM((S_PER_W, D), jnp.float32),
        pltpu.VMEM((2, S_CHUNK, D), jnp.float32),
        pltpu.SemaphoreType.DMA,
        pltpu.SemaphoreType.DMA,
        pltpu.SemaphoreType.DMA,
        pltpu.SemaphoreType.DMA,
        pltpu.SemaphoreType.DMA,
    ],
)


@jax.jit
def kernel(x, token_table, pos_emb):
    h = _sc_lookup(x.reshape(N_ROWS), token_table, pos_emb)
    return h.reshape(B, S, D)
